# Initial kernel scaffold; baseline (speedup 1.0000x reference)
import jax
import jax.numpy as jnp
from jax.experimental import pallas as pl


def kernel(input):
    raise NotImplementedError("write your pallas kernel here")


# speedup vs baseline: 2.8173x; 2.8173x over previous
import functools

import jax
import jax.numpy as jnp
from jax import lax
from jax.experimental import pallas as pl
from jax.experimental.pallas import tpu as pltpu
from jax.experimental.pallas import tpu_sc as plsc

_ROWS, _N = 64, 8192
_L = 16
_NC, _NS = 2, 16
_NW = _NC * _NS
_RPW = _ROWS // _NW
_NV = _N // _L
_BISECT_ITERS = 24


def _butterfly(v, op):
    iota = lax.iota(jnp.int32, _L)
    for k in (8, 4, 2, 1):
        v = op(v, v.at[iota ^ k].get(mode="promise_in_bounds"))
    return v


def _sc_body(x_hbm, out_hbm, buf):
    cid = lax.axis_index("c")
    sid = lax.axis_index("s")
    wid = sid * _NC + cid
    row0 = wid * _RPW
    pltpu.sync_copy(x_hbm.at[pl.ds(row0, _RPW)], buf)

    for r in range(_RPW):
        row = buf.at[r]

        def max_body(j, acc):
            return jnp.maximum(acc, row[pl.ds(j * _L, _L)])

        acc = lax.fori_loop(1, _NV, max_body, row[pl.ds(0, _L)])
        m = _butterfly(acc, jnp.maximum)

        def bis_body(i, carry):
            lo, hi = carry
            mid = 0.5 * (lo + hi)

            def inner(j, a):
                return a + jnp.maximum(row[pl.ds(j * _L, _L)] - mid, 0.0)

            a = lax.fori_loop(0, _NV, inner, jnp.zeros((_L,), jnp.float32))
            s = _butterfly(a, jnp.add)
            pred = s >= 1.0
            return jnp.where(pred, mid, lo), jnp.where(pred, hi, mid)

        lo, _hi = lax.fori_loop(0, _BISECT_ITERS, bis_body, (m - 1.0, m))

        def fin_body(j, carry):
            sa, ka = carry
            v = row[pl.ds(j * _L, _L)]
            sup = v > lo
            return (sa + jnp.where(sup, v, 0.0),
                    ka + jnp.where(sup, 1.0, 0.0))

        sa, ka = lax.fori_loop(
            0, _NV, fin_body,
            (jnp.zeros((_L,), jnp.float32), jnp.zeros((_L,), jnp.float32)))
        tau = (_butterfly(sa, jnp.add) - 1.0) / _butterfly(ka, jnp.add)

        def out_body(j, carry):
            v = row[pl.ds(j * _L, _L)]
            row[pl.ds(j * _L, _L)] = jnp.maximum(v - tau, 0.0)
            return carry

        lax.fori_loop(0, _NV, out_body, 0)

    pltpu.sync_copy(buf, out_hbm.at[pl.ds(row0, _RPW)])


@functools.partial(
    pl.kernel,
    out_type=jax.ShapeDtypeStruct((_ROWS, _N), jnp.float32),
    mesh=plsc.VectorSubcoreMesh(core_axis_name="c", subcore_axis_name="s",
                                num_cores=_NC, num_subcores=_NS),
    scratch_types=[pltpu.VMEM((_RPW, _N), jnp.float32)],
)
def _sparsemax_sc(x_hbm, out_hbm, buf):
    _sc_body(x_hbm, out_hbm, buf)


@jax.jit
def kernel(input):
    return _sparsemax_sc(input)


# speedup vs baseline: 6.2812x; 2.2295x over previous
import functools

import jax
import jax.numpy as jnp
from jax import lax
from jax.experimental import pallas as pl
from jax.experimental.pallas import tpu as pltpu
from jax.experimental.pallas import tpu_sc as plsc

_ROWS, _N = 64, 8192
_L = 16
_NC, _NS = 2, 16
_NW = _NC * _NS
_RPW = _ROWS // _NW
_NV = _N // _L
_BISECT_ITERS = 24


def _butterfly(v, op):
    iota = lax.iota(jnp.int32, _L)
    for k in (8, 4, 2, 1):
        v = op(v, v.at[iota ^ k].get(mode="promise_in_bounds"))
    return v


def _sc_body(x_hbm, out_hbm, buf):
    cid = lax.axis_index("c")
    sid = lax.axis_index("s")
    wid = sid * _NC + cid
    row0 = wid * _RPW
    pltpu.sync_copy(x_hbm.at[pl.ds(row0, _RPW)], buf)

    for r in range(_RPW):
        row = buf.at[r]

        def max_body(j, acc):
            return jnp.maximum(acc, row[pl.ds(j * _L, _L)])

        acc = lax.fori_loop(1, _NV, max_body, row[pl.ds(0, _L)], unroll=8)
        m = _butterfly(acc, jnp.maximum)

        def bis_body(i, carry):
            lo, hi = carry
            mid = 0.5 * (lo + hi)

            def inner(j, a):
                return a + jnp.maximum(row[pl.ds(j * _L, _L)] - mid, 0.0)

            a = lax.fori_loop(0, _NV, inner, jnp.zeros((_L,), jnp.float32),
                              unroll=8)
            s = _butterfly(a, jnp.add)
            pred = s >= 1.0
            return jnp.where(pred, mid, lo), jnp.where(pred, hi, mid)

        lo, _hi = lax.fori_loop(0, _BISECT_ITERS, bis_body, (m - 1.0, m))

        def fin_body(j, carry):
            sa, ka = carry
            v = row[pl.ds(j * _L, _L)]
            sup = v > lo
            return (sa + jnp.where(sup, v, 0.0),
                    ka + jnp.where(sup, 1.0, 0.0))

        sa, ka = lax.fori_loop(
            0, _NV, fin_body,
            (jnp.zeros((_L,), jnp.float32), jnp.zeros((_L,), jnp.float32)),
            unroll=8)
        tau = (_butterfly(sa, jnp.add) - 1.0) / _butterfly(ka, jnp.add)

        def out_body(j, carry):
            v = row[pl.ds(j * _L, _L)]
            row[pl.ds(j * _L, _L)] = jnp.maximum(v - tau, 0.0)
            return carry

        lax.fori_loop(0, _NV, out_body, 0, unroll=8)

    pltpu.sync_copy(buf, out_hbm.at[pl.ds(row0, _RPW)])


@functools.partial(
    pl.kernel,
    out_type=jax.ShapeDtypeStruct((_ROWS, _N), jnp.float32),
    mesh=plsc.VectorSubcoreMesh(core_axis_name="c", subcore_axis_name="s",
                                num_cores=_NC, num_subcores=_NS),
    scratch_types=[pltpu.VMEM((_RPW, _N), jnp.float32)],
)
def _sparsemax_sc(x_hbm, out_hbm, buf):
    _sc_body(x_hbm, out_hbm, buf)


@jax.jit
def kernel(input):
    return _sparsemax_sc(input)


# speedup vs baseline: 9.6701x; 1.5395x over previous
import functools

import jax
import jax.numpy as jnp
from jax import lax
from jax.experimental import pallas as pl
from jax.experimental.pallas import tpu as pltpu
from jax.experimental.pallas import tpu_sc as plsc

_ROWS, _N = 64, 8192
_L = 16
_NC, _NS = 2, 16
_NW = _NC * _NS
_RPW = _ROWS // _NW
_NV = _N // _L
_BISECT_ITERS = 16
_NACC = 4


def _butterfly(v, op):
    iota = lax.iota(jnp.int32, _L)
    for k in (8, 4, 2, 1):
        v = op(v, v.at[iota ^ k].get(mode="promise_in_bounds"))
    return v


def _sc_body(x_hbm, out_hbm, buf):
    cid = lax.axis_index("c")
    sid = lax.axis_index("s")
    wid = sid * _NC + cid
    row0 = wid * _RPW
    pltpu.sync_copy(x_hbm.at[pl.ds(row0, _RPW)], buf)

    zero = jnp.zeros((_L,), jnp.float32)

    for r in range(_RPW):
        row = buf.at[r]

        def max_body(j, accs):
            base = j * (_NACC * _L)
            return tuple(
                jnp.maximum(accs[t], row[pl.ds(base + t * _L, _L)])
                for t in range(_NACC))

        accs = lax.fori_loop(
            1, _NV // _NACC, max_body,
            tuple(row[pl.ds(t * _L, _L)] for t in range(_NACC)), unroll=4)
        m = _butterfly(jnp.maximum(jnp.maximum(accs[0], accs[1]),
                                   jnp.maximum(accs[2], accs[3])),
                       jnp.maximum)

        def bis_body(i, carry):
            lo, hi = carry
            mid = 0.5 * (lo + hi)

            def inner(j, accs):
                base = j * (_NACC * _L)
                return tuple(
                    accs[t]
                    + jnp.maximum(row[pl.ds(base + t * _L, _L)] - mid, 0.0)
                    for t in range(_NACC))

            accs = lax.fori_loop(0, _NV // _NACC, inner, (zero,) * _NACC,
                                 unroll=4)
            s = _butterfly((accs[0] + accs[1]) + (accs[2] + accs[3]), jnp.add)
            pred = s >= 1.0
            return jnp.where(pred, mid, lo), jnp.where(pred, hi, mid)

        lo, _hi = lax.fori_loop(0, _BISECT_ITERS, bis_body, (m - 1.0, m))

        def fin_body(j, carry):
            sas, kas = carry
            base = j * (_NACC * _L)
            new_s, new_k = [], []
            for t in range(_NACC):
                v = row[pl.ds(base + t * _L, _L)]
                sup = v > lo
                new_s.append(sas[t] + jnp.where(sup, v, 0.0))
                new_k.append(kas[t] + jnp.where(sup, 1.0, 0.0))
            return tuple(new_s), tuple(new_k)

        sas, kas = lax.fori_loop(
            0, _NV // _NACC, fin_body,
            ((zero,) * _NACC, (zero,) * _NACC), unroll=2)
        sa = (sas[0] + sas[1]) + (sas[2] + sas[3])
        ka = (kas[0] + kas[1]) + (kas[2] + kas[3])
        tau = (_butterfly(sa, jnp.add) - 1.0) / _butterfly(ka, jnp.add)

        def out_body(j, carry):
            base = j * (_NACC * _L)
            for t in range(_NACC):
                sl = pl.ds(base + t * _L, _L)
                row[sl] = jnp.maximum(row[sl] - tau, 0.0)
            return carry

        lax.fori_loop(0, _NV // _NACC, out_body, 0, unroll=4)

    pltpu.sync_copy(buf, out_hbm.at[pl.ds(row0, _RPW)])


@functools.partial(
    pl.kernel,
    out_type=jax.ShapeDtypeStruct((_ROWS, _N), jnp.float32),
    mesh=plsc.VectorSubcoreMesh(core_axis_name="c", subcore_axis_name="s",
                                num_cores=_NC, num_subcores=_NS),
    scratch_types=[pltpu.VMEM((_RPW, _N), jnp.float32)],
)
def _sparsemax_sc(x_hbm, out_hbm, buf):
    _sc_body(x_hbm, out_hbm, buf)


@jax.jit
def kernel(input):
    return _sparsemax_sc(input)
